# baseline (speedup 1.0000x reference)
import functools

import jax
import jax.numpy as jnp
from jax import lax
from jax.experimental import pallas as pl
from jax.experimental.pallas import tpu as pltpu
from jax.experimental.pallas import tpu_sc as plsc

UID_ALL = 1000000
IID_ALL = 1000000
EMB_DIM = 32
BATCH = 16384

_INFO = plsc.get_sparse_core_info()
NC = _INFO.num_cores
NS = _INFO.num_subcores
NW = NC * NS
BPW = BATCH // NW
CHUNK = 128
NCHUNK = BPW // CHUNK


def _body(uid_hbm, iid_hbm, xu_hbm, xi_hbm, out_hbm,
          idxu, idxi, scatu, scati, rows_u, rows_i, sem_g, sem_s):
    wid = lax.axis_index("s") * NC + lax.axis_index("c")
    base = wid * BPW

    for c in range(NCHUNK):
        pltpu.sync_copy(xu_hbm.at[wid * NCHUNK + c], idxu[c])
        pltpu.sync_copy(xi_hbm.at[wid * NCHUNK + c], idxi[c])

    gathers = []
    for c in range(NCHUNK):
        gathers.append(pltpu.async_copy(
            uid_hbm.at[idxu[c]], rows_u.at[pl.ds(c * CHUNK, CHUNK)], sem_g))
        gathers.append(pltpu.async_copy(
            iid_hbm.at[idxi[c]], rows_i.at[pl.ds(c * CHUNK, CHUNK)], sem_g))

    lane = lax.iota(jnp.int32, 16)
    for c in range(NCHUNK):
        for j in range(CHUNK // 16):
            rid = 2 * (base + c * CHUNK + j * 16 + lane)
            scatu[c][pl.ds(j * 16, 16)] = rid
            scati[c][pl.ds(j * 16, 16)] = rid + 1

    for g in gathers:
        g.wait()

    scatters = []
    for c in range(NCHUNK):
        scatters.append(pltpu.async_copy(
            rows_u.at[pl.ds(c * CHUNK, CHUNK)], out_hbm.at[scatu[c]], sem_s))
        scatters.append(pltpu.async_copy(
            rows_i.at[pl.ds(c * CHUNK, CHUNK)], out_hbm.at[scati[c]], sem_s))
    for s in scatters:
        s.wait()


@jax.jit
def kernel(x, uid_table, iid_table):
    xu = x[:, 0].reshape(BATCH // CHUNK, CHUNK)
    xi = x[:, 1].reshape(BATCH // CHUNK, CHUNK)
    mesh = plsc.VectorSubcoreMesh(core_axis_name="c", subcore_axis_name="s")
    idx_scratch = [pltpu.VMEM((CHUNK,), jnp.int32) for _ in range(NCHUNK)]
    out = pl.kernel(
        _body,
        out_type=jax.ShapeDtypeStruct((2 * BATCH, EMB_DIM), jnp.float32),
        mesh=mesh,
        compiler_params=pltpu.CompilerParams(use_tc_tiling_on_sc=False),
        scratch_types=[
            list(idx_scratch),
            list(idx_scratch),
            list(idx_scratch),
            list(idx_scratch),
            pltpu.VMEM((BPW, EMB_DIM), jnp.float32),
            pltpu.VMEM((BPW, EMB_DIM), jnp.float32),
            pltpu.SemaphoreType.DMA,
            pltpu.SemaphoreType.DMA,
        ],
    )(uid_table, iid_table, xu, xi)
    return out.reshape(BATCH, 2, EMB_DIM)
